# baseline (speedup 1.0000x reference)
import functools

import jax
import jax.numpy as jnp
from jax.experimental import pallas as pl
from jax.experimental.pallas import tpu as pltpu

_NE = 16
_TBLK = 512


def _silu(v):
    return v * jax.nn.sigmoid(v)


def _moe_body(nblk, x_ref, rw_ref, w1_ref, w3_ref, w2_ref, out_ref, s_ref):
    i = pl.program_id(0)

    @pl.when(i == 0)
    def _zero():
        s_ref[...] = jnp.zeros_like(s_ref)

    xb = x_ref[...]
    lg = jax.lax.dot_general(xb, rw_ref[...], (((1,), (1,)), ((), ())),
                             preferred_element_type=jnp.float32)
    col = jax.lax.broadcasted_iota(jnp.int32, lg.shape, 1)
    m1 = jnp.max(lg, axis=1, keepdims=True)
    i1 = jnp.min(jnp.where(lg >= m1, col, _NE), axis=1, keepdims=True)
    lm = jnp.where(col == i1, -jnp.inf, lg)
    m2 = jnp.max(lm, axis=1, keepdims=True)
    i2 = jnp.min(jnp.where(lm >= m2, col, _NE), axis=1, keepdims=True)
    g1 = jax.nn.sigmoid(m1 - m2)
    coef = (jnp.where(col == i1, g1, 0.0)
            + jnp.where(col == i2, 1.0 - g1, 0.0))

    xbf = xb.astype(jnp.bfloat16)
    a = jax.lax.dot_general(xbf, w1_ref[...], (((1,), (1,)), ((), ())),
                            preferred_element_type=jnp.float32)
    a = _silu(a).astype(jnp.bfloat16)
    h = jax.lax.dot_general(a, w3_ref[...], (((1,), (1,)), ((), ())),
                            preferred_element_type=jnp.float32)
    g = _silu(h)

    s_ref[...] += jax.lax.dot_general(coef, g, (((0,), (0,)), ((), ())),
                                      preferred_element_type=jnp.float32)

    out_ref[...] = jnp.zeros_like(out_ref)

    @pl.when(i == nblk - 1)
    def _final():
        rows = jax.lax.dot_general(s_ref[...].astype(jnp.bfloat16),
                                   w2_ref[...],
                                   (((1,), (1,)), ((), ())),
                                   preferred_element_type=jnp.float32)
        out_ref[0:_NE, :] = rows


def kernel(x, w1, w2, w3, router_w):
    b, s, d = x.shape
    xf = x.reshape(-1, d)
    n_tok = xf.shape[0]
    nblk = n_tok // _TBLK
    w1b = w1.astype(jnp.bfloat16)
    w3b = w3.astype(jnp.bfloat16)
    w2b = w2.astype(jnp.bfloat16)
    out = pl.pallas_call(
        functools.partial(_moe_body, nblk),
        grid=(nblk,),
        in_specs=[
            pl.BlockSpec((_TBLK, d), lambda i: ((i + 1) % nblk, 0)),
            pl.BlockSpec((_NE, d), lambda i: (0, 0)),
            pl.BlockSpec((d, d), lambda i: (0, 0)),
            pl.BlockSpec((d, d), lambda i: (0, 0)),
            pl.BlockSpec((d, d), lambda i: (0, 0)),
        ],
        out_specs=pl.BlockSpec((_TBLK, d), lambda i: ((i + 1) % nblk, 0)),
        out_shape=jax.ShapeDtypeStruct((n_tok, d), jnp.float32),
        scratch_shapes=[pltpu.VMEM((_NE, d), jnp.float32)],
        compiler_params=pltpu.CompilerParams(
            dimension_semantics=("arbitrary",)),
    )(xf, router_w, w1b, w3b, w2b)
    return out.reshape(b, s, d)


# speedup vs baseline: 1.1899x; 1.1899x over previous
import functools

import jax
import jax.numpy as jnp
from jax.experimental import pallas as pl
from jax.experimental.pallas import tpu as pltpu

_NE = 16
_TBLK = 512


def _silu(v):
    return v * jax.nn.sigmoid(v)


def _moe_body(nblk, x_ref, rw_ref, w1_ref, w3_ref, w2_ref, out_ref,
              s_ref, w1b_ref, w3b_ref):
    i = pl.program_id(0)

    @pl.when(i == 0)
    def _init():
        s_ref[...] = jnp.zeros_like(s_ref)
        w1b_ref[...] = w1_ref[...].astype(jnp.bfloat16)
        w3b_ref[...] = w3_ref[...].astype(jnp.bfloat16)

    xb = x_ref[...]
    lg = jax.lax.dot_general(xb, rw_ref[...], (((1,), (1,)), ((), ())),
                             preferred_element_type=jnp.float32)
    col = jax.lax.broadcasted_iota(jnp.int32, lg.shape, 1)
    m1 = jnp.max(lg, axis=1, keepdims=True)
    i1 = jnp.min(jnp.where(lg >= m1, col, _NE), axis=1, keepdims=True)
    lm = jnp.where(col == i1, -jnp.inf, lg)
    m2 = jnp.max(lm, axis=1, keepdims=True)
    i2 = jnp.min(jnp.where(lm >= m2, col, _NE), axis=1, keepdims=True)
    g1 = jax.nn.sigmoid(m1 - m2)
    coef = (jnp.where(col == i1, g1, 0.0)
            + jnp.where(col == i2, 1.0 - g1, 0.0))

    xbf = xb.astype(jnp.bfloat16)
    a = jax.lax.dot_general(xbf, w1b_ref[...], (((1,), (1,)), ((), ())),
                            preferred_element_type=jnp.float32)
    a = _silu(a).astype(jnp.bfloat16)
    h = jax.lax.dot_general(a, w3b_ref[...], (((1,), (1,)), ((), ())),
                            preferred_element_type=jnp.float32)
    g = _silu(h)

    s_ref[...] += jax.lax.dot_general(coef, g, (((0,), (0,)), ((), ())),
                                      preferred_element_type=jnp.float32)

    out_ref[...] = jnp.zeros_like(out_ref)

    @pl.when(i == nblk - 1)
    def _final():
        rows = jax.lax.dot_general(s_ref[...], w2_ref[...],
                                   (((1,), (1,)), ((), ())),
                                   preferred_element_type=jnp.float32)
        out_ref[0:_NE, :] = rows


def kernel(x, w1, w2, w3, router_w):
    b, s, d = x.shape
    xf = x.reshape(-1, d)
    n_tok = xf.shape[0]
    nblk = n_tok // _TBLK
    out = pl.pallas_call(
        functools.partial(_moe_body, nblk),
        grid=(nblk,),
        in_specs=[
            pl.BlockSpec((_TBLK, d), lambda i: ((i + 1) % nblk, 0)),
            pl.BlockSpec((_NE, d), lambda i: (0, 0)),
            pl.BlockSpec((d, d), lambda i: (0, 0)),
            pl.BlockSpec((d, d), lambda i: (0, 0)),
            pl.BlockSpec((d, d), lambda i: (0, 0)),
        ],
        out_specs=pl.BlockSpec((_TBLK, d), lambda i: ((i + 1) % nblk, 0)),
        out_shape=jax.ShapeDtypeStruct((n_tok, d), jnp.float32),
        scratch_shapes=[
            pltpu.VMEM((_NE, d), jnp.float32),
            pltpu.VMEM((d, d), jnp.bfloat16),
            pltpu.VMEM((d, d), jnp.bfloat16),
        ],
        compiler_params=pltpu.CompilerParams(
            dimension_semantics=("arbitrary",)),
    )(xf, router_w, w1, w3, w2)
    return out.reshape(b, s, d)


# speedup vs baseline: 1.1975x; 1.0064x over previous
import functools

import jax
import jax.numpy as jnp
from jax.experimental import pallas as pl
from jax.experimental.pallas import tpu as pltpu

_NE = 16
_TBLK = 1024


def _silu(v):
    return v * jax.nn.sigmoid(v)


def _moe_body(nblk, x_ref, rw_ref, w1_ref, w3_ref, w2_ref, out_ref,
              s_ref, w1b_ref, w3b_ref):
    i = pl.program_id(0)

    @pl.when(i == 0)
    def _init():
        s_ref[...] = jnp.zeros_like(s_ref)
        w1b_ref[...] = w1_ref[...].astype(jnp.bfloat16)
        w3b_ref[...] = w3_ref[...].astype(jnp.bfloat16)

    xb = x_ref[...]
    lg = jax.lax.dot_general(xb, rw_ref[...], (((1,), (1,)), ((), ())),
                             preferred_element_type=jnp.float32)
    col = jax.lax.broadcasted_iota(jnp.int32, lg.shape, 1)
    m1 = jnp.max(lg, axis=1, keepdims=True)
    i1 = jnp.min(jnp.where(lg >= m1, col, _NE), axis=1, keepdims=True)
    lm = jnp.where(col == i1, -jnp.inf, lg)
    m2 = jnp.max(lm, axis=1, keepdims=True)
    i2 = jnp.min(jnp.where(lm >= m2, col, _NE), axis=1, keepdims=True)
    g1 = jax.nn.sigmoid(m1 - m2)
    coef = (jnp.where(col == i1, g1, 0.0)
            + jnp.where(col == i2, 1.0 - g1, 0.0))

    xbf = xb.astype(jnp.bfloat16)
    a = jax.lax.dot_general(xbf, w1b_ref[...], (((1,), (1,)), ((), ())),
                            preferred_element_type=jnp.float32)
    a = _silu(a).astype(jnp.bfloat16)
    h = jax.lax.dot_general(a, w3b_ref[...], (((1,), (1,)), ((), ())),
                            preferred_element_type=jnp.float32)
    g = _silu(h)

    s_ref[...] += jax.lax.dot_general(coef, g, (((0,), (0,)), ((), ())),
                                      preferred_element_type=jnp.float32)

    out_ref[...] = jnp.zeros_like(out_ref)

    @pl.when(i == nblk - 1)
    def _final():
        rows = jax.lax.dot_general(s_ref[...], w2_ref[...],
                                   (((1,), (1,)), ((), ())),
                                   preferred_element_type=jnp.float32)
        out_ref[0:_NE, :] = rows


def kernel(x, w1, w2, w3, router_w):
    b, s, d = x.shape
    xf = x.reshape(-1, d)
    n_tok = xf.shape[0]
    nblk = n_tok // _TBLK
    out = pl.pallas_call(
        functools.partial(_moe_body, nblk),
        grid=(nblk,),
        in_specs=[
            pl.BlockSpec((_TBLK, d), lambda i: ((i + 1) % nblk, 0)),
            pl.BlockSpec((_NE, d), lambda i: (0, 0)),
            pl.BlockSpec((d, d), lambda i: (0, 0)),
            pl.BlockSpec((d, d), lambda i: (0, 0)),
            pl.BlockSpec((d, d), lambda i: (0, 0)),
        ],
        out_specs=pl.BlockSpec((_TBLK, d), lambda i: ((i + 1) % nblk, 0)),
        out_shape=jax.ShapeDtypeStruct((n_tok, d), jnp.float32),
        scratch_shapes=[
            pltpu.VMEM((_NE, d), jnp.float32),
            pltpu.VMEM((d, d), jnp.bfloat16),
            pltpu.VMEM((d, d), jnp.bfloat16),
        ],
        compiler_params=pltpu.CompilerParams(
            dimension_semantics=("arbitrary",)),
    )(xf, router_w, w1, w3, w2)
    return out.reshape(b, s, d)
